# baseline (speedup 1.0000x reference)
import functools

import jax
import jax.numpy as jnp
from jax import lax
from jax.experimental import pallas as pl
from jax.experimental.pallas import tpu as pltpu
from jax.experimental.pallas import tpu_sc as plsc

NC = 2
NS = 16
NW = NC * NS
L = 16

CHUNK = 128


def _hist_body(nch, dpad, dstp_hbm, degp_hbm, idx_v, ones_v, zb_v, acc_sh):
    c = lax.axis_index("c")
    s = lax.axis_index("s")
    gid = c * NS + s
    seg = dpad // NS
    for i in range(CHUNK // L):
        ones_v[pl.ds(i * L, L)] = jnp.ones((L,), jnp.float32)
    for i in range(seg // L):
        zb_v[pl.ds(i * L, L)] = jnp.zeros((L,), jnp.float32)
    pltpu.sync_copy(zb_v, acc_sh.at[pl.ds(s * seg, seg)])
    plsc.subcore_barrier()
    pltpu.sync_copy(dstp_hbm.at[gid], idx_v)

    def step(j, carry):
        pltpu.sync_copy(ones_v, acc_sh.at[idx_v.at[j]], add=True)
        return carry

    lax.fori_loop(0, nch, step, 0)
    plsc.subcore_barrier()
    pltpu.sync_copy(acc_sh.at[pl.ds(s * seg, seg)],
                    degp_hbm.at[c, 0, pl.ds(s * seg, seg)])


def kernel(V, E, X, W, b):
    n, d = X.shape
    e_n = E.shape[1]

    ept = -(-e_n // NW)
    nch = -(-ept // CHUNK)
    nch = nch + (nch % 2)
    total = NW * nch * CHUNK
    junk = n
    dpad = -(-(n + 1) // (NS * CHUNK)) * (NS * CHUNK)
    apad = -(-(n + 1) // (NS * 8)) * (NS * 8)

    src = E[0]
    dst = E[1]
    pad_n = total - e_n
    srcp = jnp.concatenate([src, jnp.zeros((pad_n,), jnp.int32)]).reshape(NW, nch, CHUNK)
    dstp = jnp.concatenate([dst, jnp.full((pad_n,), junk, jnp.int32)]).reshape(NW, nch, CHUNK)
    ei = jnp.stack([srcp, dstp], axis=2)

    mesh = plsc.VectorSubcoreMesh(core_axis_name="c", subcore_axis_name="s")
    hist = pl.kernel(
        functools.partial(_hist_body, nch, dpad),
        out_type=jax.ShapeDtypeStruct((NC, 1, dpad), jnp.float32),
        mesh=mesh,
        scratch_types=[
            pltpu.VMEM((nch, CHUNK), jnp.int32),
            pltpu.VMEM((CHUNK,), jnp.float32),
            pltpu.VMEM((dpad // NS,), jnp.float32),
            pltpu.VMEM_SHARED((dpad,), jnp.float32),
        ],
    )
    degp = hist(dstp)

    def _inv_body(degp_ref, inv_ref):
        dsum = degp_ref[0, 0:1, :] + degp_ref[1, 0:1, :]
        inv_ref[...] = lax.rsqrt(jnp.maximum(dsum, 1.0))

    inv_row = pl.pallas_call(
        _inv_body,
        out_shape=jax.ShapeDtypeStruct((1, dpad), jnp.float32),
    )(degp)
    inv_col = inv_row.reshape(dpad, 1)[:n]

    rb = 1000
    grid = n // rb

    def _mm_body(x_ref, w_ref, inv_ref, ys_ref):
        ys_ref[...] = jnp.dot(x_ref[...], w_ref[...],
                              preferred_element_type=jnp.float32) * inv_ref[...]

    ys = pl.pallas_call(
        _mm_body,
        grid=(grid,),
        in_specs=[
            pl.BlockSpec((rb, d), lambda i: (i, 0)),
            pl.BlockSpec((d, d), lambda i: (0, 0)),
            pl.BlockSpec((rb, 1), lambda i: (i, 0)),
        ],
        out_specs=pl.BlockSpec((rb, d), lambda i: (i, 0)),
        out_shape=jax.ShapeDtypeStruct((n, d), jnp.float32),
    )(X, W, inv_col)

    assert nch % 2 == 0

    def _agg(ys_hbm, ei_hbm, accp_hbm, iv0, iv1, rows0, rows1, acc_sh,
             isem0, isem1, gsem0, gsem1):
        c = lax.axis_index("c")
        s = lax.axis_index("s")
        gid = c * NS + s
        z16 = jnp.zeros((L,), jnp.float32)

        def zrow(r, carry):
            for cc in range(d // L):
                rows0[r, pl.ds(cc * L, L)] = z16
            return carry

        lax.fori_loop(0, CHUNK, zrow, 0)

        zseg = apad // NS
        off = 0
        while off < zseg:
            sz = min(CHUNK, zseg - off)
            pltpu.sync_copy(rows0.at[pl.ds(0, sz)],
                            acc_sh.at[pl.ds(s * zseg + off, sz)])
            off += sz

        pltpu.sync_copy(ei_hbm.at[gid, 0], iv0)
        pltpu.async_copy(ei_hbm.at[gid, 1], iv1, isem1)
        pltpu.async_copy(ys_hbm.at[iv0.at[0]], rows0, gsem0)
        plsc.subcore_barrier()

        def step(jj, carry):
            j = jj * 2
            pltpu.make_async_copy(ei_hbm.at[gid, 0], iv1, isem1).wait()
            pltpu.make_async_copy(ys_hbm.at[iv0.at[0]], rows0, gsem0).wait()
            pltpu.async_copy(ys_hbm.at[iv1.at[0]], rows1, gsem1)
            pltpu.sync_copy(rows0, acc_sh.at[iv0.at[1]], add=True)

            @pl.when(j + 2 < nch)
            def _():
                pltpu.async_copy(ei_hbm.at[gid, j + 2], iv0, isem0)

            pltpu.make_async_copy(ys_hbm.at[iv1.at[0]], rows1, gsem1).wait()

            @pl.when(j + 2 < nch)
            def _():
                pltpu.make_async_copy(ei_hbm.at[gid, 0], iv0, isem0).wait()
                pltpu.async_copy(ys_hbm.at[iv0.at[0]], rows0, gsem0)

            pltpu.sync_copy(rows1, acc_sh.at[iv1.at[1]], add=True)

            @pl.when(j + 3 < nch)
            def _():
                pltpu.async_copy(ei_hbm.at[gid, j + 3], iv1, isem1)

            return carry

        lax.fori_loop(0, nch // 2, step, 0)
        plsc.subcore_barrier()

        wseg = apad // NS
        pltpu.sync_copy(acc_sh.at[pl.ds(s * wseg, wseg)],
                        accp_hbm.at[c, pl.ds(s * wseg, wseg)])

    agg = pl.kernel(
        _agg,
        out_type=jax.ShapeDtypeStruct((NC, apad, d), jnp.float32),
        mesh=mesh,
        scratch_types=[
            pltpu.VMEM((2, CHUNK), jnp.int32),
            pltpu.VMEM((2, CHUNK), jnp.int32),
            pltpu.VMEM((CHUNK, d), jnp.float32),
            pltpu.VMEM((CHUNK, d), jnp.float32),
            pltpu.VMEM_SHARED((apad, d), jnp.float32),
            pltpu.SemaphoreType.DMA,
            pltpu.SemaphoreType.DMA,
            pltpu.SemaphoreType.DMA,
            pltpu.SemaphoreType.DMA,
        ],
    )
    accp = agg(ys, ei)

    def _fin_body(accp_ref, inv_ref, b_ref, out_ref):
        a = accp_ref[0] + accp_ref[1]
        out_ref[...] = a[:out_ref.shape[0]] * inv_ref[...] + b_ref[...]

    out = pl.pallas_call(
        _fin_body,
        out_shape=jax.ShapeDtypeStruct((n, d), jnp.float32),
    )(accp, inv_col, b.reshape(1, d))
    return out


# speedup vs baseline: 2.8888x; 2.8888x over previous
import functools

import jax
import jax.numpy as jnp
from jax import lax
from jax.experimental import pallas as pl
from jax.experimental.pallas import tpu as pltpu
from jax.experimental.pallas import tpu_sc as plsc

NC = 2
NS = 16
NW = NC * NS
L = 16

CHUNK = 128


def _hist_body(nch, dpad, dstp_hbm, degp_hbm, idx_v, ones_v, zb_v, acc_sh):
    c = lax.axis_index("c")
    s = lax.axis_index("s")
    gid = c * NS + s
    seg = dpad // NS
    for i in range(CHUNK // L):
        ones_v[pl.ds(i * L, L)] = jnp.ones((L,), jnp.float32)
    for i in range(seg // L):
        zb_v[pl.ds(i * L, L)] = jnp.zeros((L,), jnp.float32)
    pltpu.sync_copy(zb_v, acc_sh.at[pl.ds(s * seg, seg)])
    plsc.subcore_barrier()
    pltpu.sync_copy(dstp_hbm.at[gid], idx_v)

    def step(j, carry):
        pltpu.sync_copy(ones_v, acc_sh.at[idx_v.at[j]], add=True)
        return carry

    lax.fori_loop(0, nch, step, 0)
    plsc.subcore_barrier()
    pltpu.sync_copy(acc_sh.at[pl.ds(s * seg, seg)],
                    degp_hbm.at[c, 0, pl.ds(s * seg, seg)])


def kernel(V, E, X, W, b):
    n, d = X.shape
    e_n = E.shape[1]

    ept = -(-e_n // NW)
    nch = -(-ept // CHUNK)
    nch = nch + (nch % 2)
    total = NW * nch * CHUNK
    junk = n
    dpad = -(-(n + 1) // (NS * CHUNK)) * (NS * CHUNK)
    apad = -(-(n + 1) // (NS * 8)) * (NS * 8)

    src = E[0]
    dst = E[1]
    pad_n = total - e_n
    pr = jnp.arange(pad_n, dtype=jnp.int32)
    srcp = jnp.concatenate([src, pr % n]).reshape(nch, NW, CHUNK)
    srcp = srcp.transpose(1, 0, 2)
    dstp = jnp.concatenate([dst, junk + pr % (apad - n)]).reshape(nch, NW, CHUNK)
    dstp = dstp.transpose(1, 0, 2)
    ei = jnp.stack([srcp, dstp], axis=2)

    mesh = plsc.VectorSubcoreMesh(core_axis_name="c", subcore_axis_name="s")
    hist = pl.kernel(
        functools.partial(_hist_body, nch, dpad),
        out_type=jax.ShapeDtypeStruct((NC, 1, dpad), jnp.float32),
        mesh=mesh,
        scratch_types=[
            pltpu.VMEM((nch, CHUNK), jnp.int32),
            pltpu.VMEM((CHUNK,), jnp.float32),
            pltpu.VMEM((dpad // NS,), jnp.float32),
            pltpu.VMEM_SHARED((dpad,), jnp.float32),
        ],
    )
    degp = hist(dstp)

    def _inv_body(degp_ref, inv_ref):
        dsum = degp_ref[0, 0:1, :] + degp_ref[1, 0:1, :]
        inv_ref[...] = lax.rsqrt(jnp.maximum(dsum, 1.0))

    inv_row = pl.pallas_call(
        _inv_body,
        out_shape=jax.ShapeDtypeStruct((1, dpad), jnp.float32),
    )(degp)
    inv_col = inv_row.reshape(dpad, 1)[:n]

    rb = 1000
    grid = n // rb

    def _mm_body(x_ref, w_ref, inv_ref, ys_ref):
        ys_ref[...] = jnp.dot(x_ref[...], w_ref[...],
                              preferred_element_type=jnp.float32) * inv_ref[...]

    ys = pl.pallas_call(
        _mm_body,
        grid=(grid,),
        in_specs=[
            pl.BlockSpec((rb, d), lambda i: (i, 0)),
            pl.BlockSpec((d, d), lambda i: (0, 0)),
            pl.BlockSpec((rb, 1), lambda i: (i, 0)),
        ],
        out_specs=pl.BlockSpec((rb, d), lambda i: (i, 0)),
        out_shape=jax.ShapeDtypeStruct((n, d), jnp.float32),
    )(X, W, inv_col)

    assert nch % 2 == 0

    def _agg(ys_hbm, ei_hbm, accp_hbm, iv0, iv1, rows0, rows1, acc_sh,
             isem0, isem1, gsem0, gsem1):
        c = lax.axis_index("c")
        s = lax.axis_index("s")
        gid = c * NS + s
        z16 = jnp.zeros((L,), jnp.float32)

        def zrow(r, carry):
            for cc in range(d // L):
                rows0[r, pl.ds(cc * L, L)] = z16
            return carry

        lax.fori_loop(0, CHUNK, zrow, 0)

        zseg = apad // NS
        off = 0
        while off < zseg:
            sz = min(CHUNK, zseg - off)
            pltpu.sync_copy(rows0.at[pl.ds(0, sz)],
                            acc_sh.at[pl.ds(s * zseg + off, sz)])
            off += sz

        pltpu.sync_copy(ei_hbm.at[gid, 0], iv0)
        pltpu.async_copy(ei_hbm.at[gid, 1], iv1, isem1)
        pltpu.async_copy(ys_hbm.at[iv0.at[0]], rows0, gsem0)
        plsc.subcore_barrier()

        def step(jj, carry):
            j = jj * 2
            pltpu.make_async_copy(ei_hbm.at[gid, 0], iv1, isem1).wait()
            pltpu.make_async_copy(ys_hbm.at[iv0.at[0]], rows0, gsem0).wait()
            pltpu.async_copy(ys_hbm.at[iv1.at[0]], rows1, gsem1)
            pltpu.sync_copy(rows0, acc_sh.at[iv0.at[1]], add=True)

            @pl.when(j + 2 < nch)
            def _():
                pltpu.async_copy(ei_hbm.at[gid, j + 2], iv0, isem0)

            pltpu.make_async_copy(ys_hbm.at[iv1.at[0]], rows1, gsem1).wait()

            @pl.when(j + 2 < nch)
            def _():
                pltpu.make_async_copy(ei_hbm.at[gid, 0], iv0, isem0).wait()
                pltpu.async_copy(ys_hbm.at[iv0.at[0]], rows0, gsem0)

            pltpu.sync_copy(rows1, acc_sh.at[iv1.at[1]], add=True)

            @pl.when(j + 3 < nch)
            def _():
                pltpu.async_copy(ei_hbm.at[gid, j + 3], iv1, isem1)

            return carry

        lax.fori_loop(0, nch // 2, step, 0)
        plsc.subcore_barrier()

        wseg = apad // NS
        pltpu.sync_copy(acc_sh.at[pl.ds(s * wseg, wseg)],
                        accp_hbm.at[c, pl.ds(s * wseg, wseg)])

    agg = pl.kernel(
        _agg,
        out_type=jax.ShapeDtypeStruct((NC, apad, d), jnp.float32),
        mesh=mesh,
        scratch_types=[
            pltpu.VMEM((2, CHUNK), jnp.int32),
            pltpu.VMEM((2, CHUNK), jnp.int32),
            pltpu.VMEM((CHUNK, d), jnp.float32),
            pltpu.VMEM((CHUNK, d), jnp.float32),
            pltpu.VMEM_SHARED((apad, d), jnp.float32),
            pltpu.SemaphoreType.DMA,
            pltpu.SemaphoreType.DMA,
            pltpu.SemaphoreType.DMA,
            pltpu.SemaphoreType.DMA,
        ],
    )
    accp = agg(ys, ei)

    def _fin_body(accp_ref, inv_ref, b_ref, out_ref):
        a = accp_ref[0] + accp_ref[1]
        out_ref[...] = a[:out_ref.shape[0]] * inv_ref[...] + b_ref[...]

    out = pl.pallas_call(
        _fin_body,
        out_shape=jax.ShapeDtypeStruct((n, d), jnp.float32),
    )(accp, inv_col, b.reshape(1, d))
    return out
